# baseline (speedup 1.0000x reference)
import functools

import jax
import jax.numpy as jnp
from jax import lax
from jax.experimental import pallas as pl
from jax.experimental.pallas import tpu as pltpu
from jax.experimental.pallas import tpu_sc as plsc

_N0 = 16384
_N1 = 26
_N1P = 32
_DP = 128
_B = _N0 * _N1P
_V = 1000000
_TCOLS = 7813
_NC = 2
_NS = 16
_NW = _NC * _NS
_B_PER_W = _B // _NW
_CHUNK = 256
_N_CHUNKS = _B_PER_W // _CHUNK
_NBUF = 2
_NGROUPS = _N_CHUNKS // _NBUF

_mesh = plsc.VectorSubcoreMesh(core_axis_name="c", subcore_axis_name="s")


@functools.partial(
    pl.kernel,
    mesh=_mesh,
    out_type=jax.ShapeDtypeStruct((_V, _DP), jnp.float32),
    scratch_types=[
        pltpu.VMEM((2, 64, 128), jnp.float32),
        pltpu.VMEM((2, 128, 128), jnp.float32),
        pltpu.SemaphoreType.DMA,
        pltpu.SemaphoreType.DMA,
        pltpu.SemaphoreType.DMA,
        pltpu.SemaphoreType.DMA,
    ],
    compiler_params=pltpu.CompilerParams(
        use_tc_tiling_on_sc=True,
        disable_bounds_checks=True,
        needs_layout_passes=False,
    ),
)
def _transpose_kernel(tt_hbm, out_hbm, a_v, b_v, *sems):
    asem = sems[:2]
    bsem = sems[2:]
    wid = lax.axis_index("s") * _NC + lax.axis_index("c")
    lo = wid * _TCOLS // _NW
    hi = (wid + 1) * _TCOLS // _NW
    iota = lax.iota(jnp.int32, 16)
    _rots = [(iota + m) % 16 for m in range(16)]

    def a_copy(t, p):
        return pltpu.make_async_copy(
            tt_hbm.at[:, pl.ds(t * 128, 128)], a_v.at[p], asem[p]
        )

    def b_copy_full(t, p):
        return pltpu.make_async_copy(
            b_v.at[p], out_hbm.at[pl.ds(t * 128, 128)], bsem[p]
        )

    def b_copy_last(t, p):
        return pltpu.make_async_copy(
            b_v.at[p, pl.ds(0, 64)], out_hbm.at[pl.ds(t * 128, 64)], bsem[p]
        )

    a_copy(lo, 0).start()

    @pl.when(lo + 1 < hi)
    def _():
        a_copy(lo + 1, 1).start()

    def body(q, carry):
        for b in (0, 1):
            t = lo + 2 * q + b

            @pl.when(t < hi)
            def _():
                a_copy(t, b).wait()

                @pl.when(t - 2 >= lo)
                def _():
                    b_copy_full(t - 2, b).wait()

                a2 = a_v.at[b]
                b2 = b_v.at[b]

                def krow(k4, c2):
                    rk = k4 * 16 + iota
                    for c0 in range(0, 128, 16):
                        vs = []
                        for m in range(16):
                            cols = c0 + _rots[m]
                            vs.append((cols, plsc.load_gather(a2, [rk, cols])))
                        for cols, v in vs:
                            plsc.store_scatter(b2, [cols, rk], v)
                    return c2

                lax.fori_loop(0, 4, krow, 0)

                @pl.when(t + 2 < hi)
                def _():
                    a_copy(t + 2, b).start()

                @pl.when(t == _TCOLS - 1)
                def _():
                    b_copy_last(t, b).start()
                    b_copy_last(t, b).wait()

                @pl.when(t < _TCOLS - 1)
                def _():
                    b_copy_full(t, b).start()

        return carry

    lax.fori_loop(0, (hi - lo + 1) // 2, body, 0)

    for b in (0, 1):
        for t_off in (1, 2):
            t = hi - t_off

            @pl.when(
                ((t - lo) % 2 == b) & (t >= lo) & (t < _TCOLS - 1)
            )
            def _():
                b_copy_full(t, b).wait()


@functools.partial(
    pl.kernel,
    mesh=_mesh,
    out_type=jax.ShapeDtypeStruct((_B, _DP), jnp.float32),
    scratch_types=[
        pltpu.VMEM((_B_PER_W,), jnp.int32),
        pltpu.VMEM((_NBUF, _CHUNK, _DP), jnp.float32),
    ]
    + [pltpu.SemaphoreType.DMA] * (2 * _NBUF),
    compiler_params=pltpu.CompilerParams(use_tc_tiling_on_sc=True),
)
def _gather_kernel(idx_hbm, table_hbm, out_hbm, idx_v, rows_v, *sems):
    gsem = sems[:_NBUF]
    osem = sems[_NBUF:]
    wid = lax.axis_index("s") * _NC + lax.axis_index("c")
    base = wid * _B_PER_W
    pltpu.sync_copy(idx_hbm.at[pl.ds(base, _B_PER_W)], idx_v)

    def g_copy(ci, b):
        return pltpu.make_async_copy(
            table_hbm.at[idx_v.at[pl.ds(ci * _CHUNK, _CHUNK)]],
            rows_v.at[b],
            gsem[b],
        )

    def o_copy(ci, b):
        return pltpu.make_async_copy(
            rows_v.at[b],
            out_hbm.at[pl.ds(base + ci * _CHUNK, _CHUNK)],
            osem[b],
        )

    for b in range(_NBUF):
        g_copy(b, b).start()

    def body(g, carry):
        ci0 = g * _NBUF
        for b in range(_NBUF):
            g_copy(ci0 + b, b).wait()
            o_copy(ci0 + b, b).start()
        for b in range(_NBUF):
            o_copy(ci0 + b, b).wait()
            g_copy(ci0 + _NBUF + b, b).start()
        return carry

    lax.fori_loop(0, _NGROUPS - 1, body, 0)

    ci0 = (_NGROUPS - 1) * _NBUF
    for b in range(_NBUF):
        g_copy(ci0 + b, b).wait()
        o_copy(ci0 + b, b).start()
    for b in range(_NBUF):
        o_copy(ci0 + b, b).wait()


def kernel(idx, action_embedding):
    table128 = _transpose_kernel(action_embedding.T)
    junk = jnp.broadcast_to(
        (jnp.arange(_N0, dtype=idx.dtype) % _V)[:, None], (_N0, _N1P - _N1)
    )
    idx_pad = jnp.concatenate([idx, junk], axis=1).reshape(-1)
    flat = _gather_kernel(idx_pad, table128)
    return flat.reshape(_N0, _N1P, _DP)[:, :_N1, :64]


# speedup vs baseline: 1.0137x; 1.0137x over previous
import functools

import jax
import jax.numpy as jnp
from jax import lax
from jax.experimental import pallas as pl
from jax.experimental.pallas import tpu as pltpu
from jax.experimental.pallas import tpu_sc as plsc

_N0 = 16384
_N1 = 26
_N1P = 32
_DP = 128
_B = _N0 * _N1P
_V = 1000000
_TCOLS = 7813
_NC = 2
_NS = 16
_NW = _NC * _NS
_B_PER_W = _B // _NW
_CHUNK = 128
_N_CHUNKS = _B_PER_W // _CHUNK
_NBUF = 4
_NGROUPS = _N_CHUNKS // _NBUF

_mesh = plsc.VectorSubcoreMesh(core_axis_name="c", subcore_axis_name="s")


@functools.partial(
    pl.kernel,
    mesh=_mesh,
    out_type=jax.ShapeDtypeStruct((_V, _DP), jnp.float32),
    scratch_types=[
        pltpu.VMEM((2, 64, 128), jnp.float32),
        pltpu.VMEM((2, 128, 128), jnp.float32),
        pltpu.SemaphoreType.DMA,
        pltpu.SemaphoreType.DMA,
        pltpu.SemaphoreType.DMA,
        pltpu.SemaphoreType.DMA,
    ],
    compiler_params=pltpu.CompilerParams(
        use_tc_tiling_on_sc=True,
        disable_bounds_checks=True,
        needs_layout_passes=False,
    ),
)
def _transpose_kernel(tt_hbm, out_hbm, a_v, b_v, *sems):
    asem = sems[:2]
    bsem = sems[2:]
    wid = lax.axis_index("s") * _NC + lax.axis_index("c")
    lo = wid * _TCOLS // _NW
    hi = (wid + 1) * _TCOLS // _NW
    iota = lax.iota(jnp.int32, 16)
    _rots = [(iota + m) % 16 for m in range(16)]

    def a_copy(t, p):
        return pltpu.make_async_copy(
            tt_hbm.at[:, pl.ds(t * 128, 128)], a_v.at[p], asem[p]
        )

    def b_copy_full(t, p):
        return pltpu.make_async_copy(
            b_v.at[p], out_hbm.at[pl.ds(t * 128, 128)], bsem[p]
        )

    def b_copy_last(t, p):
        return pltpu.make_async_copy(
            b_v.at[p, pl.ds(0, 64)], out_hbm.at[pl.ds(t * 128, 64)], bsem[p]
        )

    a_copy(lo, 0).start()

    @pl.when(lo + 1 < hi)
    def _():
        a_copy(lo + 1, 1).start()

    def body(q, carry):
        for b in (0, 1):
            t = lo + 2 * q + b

            @pl.when(t < hi)
            def _():
                a_copy(t, b).wait()

                @pl.when(t - 2 >= lo)
                def _():
                    b_copy_full(t - 2, b).wait()

                a2 = a_v.at[b]
                b2 = b_v.at[b]

                def krow(k4, c2):
                    rk = k4 * 16 + iota
                    for c0 in range(0, 128, 16):
                        vs = []
                        for m in range(16):
                            cols = c0 + _rots[m]
                            vs.append((cols, plsc.load_gather(a2, [rk, cols])))
                        for cols, v in vs:
                            plsc.store_scatter(b2, [cols, rk], v)
                    return c2

                lax.fori_loop(0, 4, krow, 0)

                @pl.when(t + 2 < hi)
                def _():
                    a_copy(t + 2, b).start()

                @pl.when(t == _TCOLS - 1)
                def _():
                    b_copy_last(t, b).start()
                    b_copy_last(t, b).wait()

                @pl.when(t < _TCOLS - 1)
                def _():
                    b_copy_full(t, b).start()

        return carry

    lax.fori_loop(0, (hi - lo + 1) // 2, body, 0)

    for b in (0, 1):
        for t_off in (1, 2):
            t = hi - t_off

            @pl.when(
                ((t - lo) % 2 == b) & (t >= lo) & (t < _TCOLS - 1)
            )
            def _():
                b_copy_full(t, b).wait()


@functools.partial(
    pl.kernel,
    mesh=_mesh,
    out_type=jax.ShapeDtypeStruct((_B, _DP), jnp.float32),
    scratch_types=[
        pltpu.VMEM((_B_PER_W,), jnp.int32),
        pltpu.VMEM((_NBUF, _CHUNK, _DP), jnp.float32),
    ]
    + [pltpu.SemaphoreType.DMA] * (2 * _NBUF),
    compiler_params=pltpu.CompilerParams(use_tc_tiling_on_sc=True),
)
def _gather_kernel(idx_hbm, table_hbm, out_hbm, idx_v, rows_v, *sems):
    gsem = sems[:_NBUF]
    osem = sems[_NBUF:]
    wid = lax.axis_index("s") * _NC + lax.axis_index("c")
    base = wid * _B_PER_W
    pltpu.sync_copy(idx_hbm.at[pl.ds(base, _B_PER_W)], idx_v)

    def g_copy(ci, b):
        return pltpu.make_async_copy(
            table_hbm.at[idx_v.at[pl.ds(ci * _CHUNK, _CHUNK)]],
            rows_v.at[b],
            gsem[b],
        )

    def o_copy(ci, b):
        return pltpu.make_async_copy(
            rows_v.at[b],
            out_hbm.at[pl.ds(base + ci * _CHUNK, _CHUNK)],
            osem[b],
        )

    for b in range(_NBUF):
        g_copy(b, b).start()

    def body(g, carry):
        ci0 = g * _NBUF
        for b in range(_NBUF):
            g_copy(ci0 + b, b).wait()
            o_copy(ci0 + b, b).start()
        for b in range(_NBUF):
            o_copy(ci0 + b, b).wait()
            g_copy(ci0 + _NBUF + b, b).start()
        return carry

    lax.fori_loop(0, _NGROUPS - 1, body, 0)

    ci0 = (_NGROUPS - 1) * _NBUF
    for b in range(_NBUF):
        g_copy(ci0 + b, b).wait()
        o_copy(ci0 + b, b).start()
    for b in range(_NBUF):
        o_copy(ci0 + b, b).wait()


def kernel(idx, action_embedding):
    table128 = _transpose_kernel(action_embedding.T)
    junk = jnp.broadcast_to(
        (jnp.arange(_N0, dtype=idx.dtype) % _V)[:, None], (_N0, _N1P - _N1)
    )
    idx_pad = jnp.concatenate([idx, junk], axis=1).reshape(-1)
    flat = _gather_kernel(idx_pad, table128)
    return flat.reshape(_N0, _N1P, _DP)[:, :_N1, :64]
